# baseline (speedup 1.0000x reference)
import functools

import jax
import jax.numpy as jnp
from jax import lax
from jax.experimental import pallas as pl
from jax.experimental.pallas import tpu as pltpu
from jax.experimental.pallas import tpu_sc as plsc

N = 10000
E = 320000
D = 128
DH = D // 2

NC = 2
NS = 16
NW = NC * NS

CH = 128
EPAD = 327680
NROW = EPAD // CH
CHT = NROW // NS
NPAD = 10112
ROWS_PER_TILE = NPAD // NS

MM_BLK = 1000


def _mm_body(x_ref, w0_ref, w1_ref, b0_ref, b1_ref, o0_ref, o1_ref):
    dn = (((1,), (1,)), ((), ()))
    o0_ref[...] = lax.dot_general(
        x_ref[...], w0_ref[...], dimension_numbers=dn,
        preferred_element_type=jnp.float32) + b0_ref[...]
    o1_ref[...] = lax.dot_general(
        x_ref[...], w1_ref[...], dimension_numbers=dn,
        preferred_element_type=jnp.float32) + b1_ref[...]


def _support_matmul(x, W, b):
    return pl.pallas_call(
        _mm_body,
        grid=(N // MM_BLK,),
        in_specs=[
            pl.BlockSpec((MM_BLK, D), lambda i: (i, 0)),
            pl.BlockSpec((DH, D), lambda i: (0, 0)),
            pl.BlockSpec((DH, D), lambda i: (0, 0)),
            pl.BlockSpec((1, DH), lambda i: (0, 0)),
            pl.BlockSpec((1, DH), lambda i: (0, 0)),
        ],
        out_specs=[
            pl.BlockSpec((MM_BLK, DH), lambda i: (i, 0)),
            pl.BlockSpec((MM_BLK, DH), lambda i: (i, 0)),
        ],
        out_shape=[
            jax.ShapeDtypeStruct((N, DH), jnp.float32),
            jax.ShapeDtypeStruct((N, DH), jnp.float32),
        ],
    )(x, W[:DH], W[DH:], b[:DH].reshape(1, DH), b[DH:].reshape(1, DH))


def _edge_body(sup0_hbm, sup1_hbm, ed_hbm, zero_hbm, out_hbm,
               e0, e1, e2, e3, rows0, rows1, rows2, rows3, agg,
               isem0, isem1, isem2, isem3,
               gsem0, gsem1, gsem2, gsem3):
    ebufs = (e0, e1, e2, e3)
    isems = (isem0, isem1, isem2, isem3)
    gsems = (gsem0, gsem1, gsem2, gsem3)
    rows = (rows0, rows1, rows2, rows3)

    cid = lax.axis_index("c")
    sid = lax.axis_index("s")
    base = sid * CHT

    r0 = sid * ROWS_PER_TILE
    pltpu.sync_copy(zero_hbm, agg.at[pl.ds(r0, ROWS_PER_TILE)])
    plsc.subcore_barrier()

    def edge_load(t, s):
        return pltpu.make_async_copy(ed_hbm.at[base + t], ebufs[s], isems[s])

    def gather_start(s):
        @pl.when(cid == 0)
        def _():
            pltpu.make_async_copy(
                sup0_hbm.at[ebufs[s].at[0]], rows[s], gsems[s]).start()

        @pl.when(cid == 1)
        def _():
            pltpu.make_async_copy(
                sup1_hbm.at[ebufs[s].at[0]], rows[s], gsems[s]).start()

    def gather_wait(s):
        pltpu.make_async_copy(
            sup0_hbm.at[ebufs[s].at[0]], rows[s], gsems[s]).wait()

    for s in range(4):
        edge_load(s, s).start()
    for s in range(2):
        edge_load(s, s).wait()
        gather_start(s)

    def body(g, carry):
        for k in range(4):
            t = g * 4 + k
            k2 = (k + 2) % 4
            last_g = CHT // 4 - 1

            def prefetch_gather():
                edge_load(t + 2, k2).wait()
                gather_start(k2)

            if k < 2:
                prefetch_gather()
            else:
                pl.when(g < last_g)(prefetch_gather)

            gather_wait(k)

            cur = rows[k]
            eb = ebufs[k]

            def scale_body(grp, c2):
                w16 = lax.bitcast_convert_type(
                    eb[2, pl.ds(grp * 16, 16)], jnp.float32)
                for l in range(16):
                    w = w16[l]
                    e = grp * 16 + l
                    for j in range(DH // 16):
                        sl = pl.ds(j * 16, 16)
                        cur[e, sl] = cur[e, sl] * w
                return c2

            lax.fori_loop(0, CH // 16, scale_body, 0)

            pltpu.sync_copy(cur, agg.at[eb.at[1]], add=True)

            def prefetch_edges():
                edge_load(t + 4, k).start()

            pl.when(g < last_g)(prefetch_edges)
        return carry

    lax.fori_loop(0, CHT // 4, body, 0)
    plsc.subcore_barrier()

    pltpu.sync_copy(agg.at[pl.ds(r0, ROWS_PER_TILE)],
                    out_hbm.at[cid, pl.ds(r0, ROWS_PER_TILE)])


_edge_kernel = functools.partial(
    pl.kernel,
    out_type=jax.ShapeDtypeStruct((NC, NPAD, DH), jnp.float32),
    mesh=plsc.VectorSubcoreMesh(core_axis_name="c", subcore_axis_name="s"),
    compiler_params=pltpu.CompilerParams(use_tc_tiling_on_sc=False),
    scratch_types=(
        [pltpu.VMEM((3, CH), jnp.int32)] * 4
        + [pltpu.VMEM((CH, DH), jnp.float32)] * 4
        + [pltpu.VMEM_SHARED((NPAD, DH), jnp.float32)]
        + [pltpu.SemaphoreType.DMA] * 8
    ),
)(_edge_body)


def _comb_body(p_ref, o_ref):
    o_ref[...] = jnp.tanh(jnp.concatenate([p_ref[0], p_ref[1]], axis=1))


def _combine(partials):
    return pl.pallas_call(
        _comb_body,
        grid=(N // MM_BLK,),
        in_specs=[pl.BlockSpec((NC, MM_BLK, DH), lambda i: (0, i, 0))],
        out_specs=pl.BlockSpec((MM_BLK, D), lambda i: (i, 0)),
        out_shape=jax.ShapeDtypeStruct((N, D), jnp.float32),
    )(partials)


def kernel(x, edge_index, edge_weight, W, b):
    dst = edge_index[0].astype(jnp.int32)
    src = edge_index[1].astype(jnp.int32)
    npad = EPAD - E
    pad_idx = jnp.arange(npad, dtype=jnp.int32) % N
    src2 = jnp.concatenate([src, pad_idx]).reshape(NROW, CH)
    dst2 = jnp.concatenate([dst, pad_idx]).reshape(NROW, CH)
    wb2 = jnp.concatenate(
        [edge_weight.view(jnp.int32),
         jnp.zeros((npad,), jnp.int32)]).reshape(NROW, CH)
    edata = jnp.stack([src2, dst2, wb2], axis=1)
    sup0, sup1 = _support_matmul(x, W, b)
    zeros = jnp.zeros((ROWS_PER_TILE, DH), jnp.float32)
    partials = _edge_kernel(sup0, sup1, edata, zeros)
    return _combine(partials)


# speedup vs baseline: 1.3603x; 1.3603x over previous
import functools

import jax
import jax.numpy as jnp
from jax import lax
from jax.experimental import pallas as pl
from jax.experimental.pallas import tpu as pltpu
from jax.experimental.pallas import tpu_sc as plsc

N = 10000
E = 320000
D = 128

NC = 2
NS = 16
NW = NC * NS

CH = 112
CHW = 90
NROW = NW * CHW
EPAD = NROW * CH
NPAD = 10112
ROWS_PER_TILE = NPAD // NS

NRB = 3
NEB = 6
UNROLL = 6

MM_BLK = 1000


def _mm_body(x_ref, w_ref, b_ref, o_ref):
    o_ref[...] = lax.dot_general(
        x_ref[...], w_ref[...],
        dimension_numbers=(((1,), (1,)), ((), ())),
        preferred_element_type=jnp.float32,
    ) + b_ref[...]


def _support_matmul(x, W, b2):
    return pl.pallas_call(
        _mm_body,
        grid=(N // MM_BLK,),
        in_specs=[
            pl.BlockSpec((MM_BLK, D), lambda i: (i, 0)),
            pl.BlockSpec((D, D), lambda i: (0, 0)),
            pl.BlockSpec((1, D), lambda i: (0, 0)),
        ],
        out_specs=pl.BlockSpec((MM_BLK, D), lambda i: (i, 0)),
        out_shape=jax.ShapeDtypeStruct((N, D), jnp.float32),
    )(x, W, b2)


def _edge_body(sup_hbm, ed_hbm, zero_hbm, out_hbm,
               e0, e1, e2, e3, e4, e5, rows0, rows1, rows2, agg,
               isem0, isem1, isem2, isem3, isem4, isem5,
               gsem0, gsem1, gsem2, ssem0, ssem1, ssem2):
    ebufs = (e0, e1, e2, e3, e4, e5)
    isems = (isem0, isem1, isem2, isem3, isem4, isem5)
    gsems = (gsem0, gsem1, gsem2)
    ssems = (ssem0, ssem1, ssem2)
    rows = (rows0, rows1, rows2)

    cid = lax.axis_index("c")
    sid = lax.axis_index("s")
    wid = sid * NC + cid
    base = wid * CHW

    r0 = sid * ROWS_PER_TILE
    pltpu.sync_copy(zero_hbm, agg.at[pl.ds(r0, ROWS_PER_TILE)])
    plsc.subcore_barrier()

    def edge_load(t, s):
        return pltpu.make_async_copy(ed_hbm.at[base + t], ebufs[s], isems[s])

    def gather(s, r):
        return pltpu.make_async_copy(
            sup_hbm.at[ebufs[s].at[0]], rows[r], gsems[r])

    def scatter_desc(s, r):
        return pltpu.make_async_copy(rows[r], agg.at[ebufs[s].at[1]],
                                     ssems[r])

    for s in range(3):
        edge_load(s, s).start()
    edge_load(0, 0).wait()
    gather(0, 0).start()

    def body(g, carry):
        for k in range(UNROLL):
            t = g * UNROLL + k
            r = k % NRB
            rn = (k + 1) % NRB
            kn = (k + 1) % NEB
            last_g = CHW // UNROLL - 1

            def prefetch_gather():
                edge_load(t + 1, kn).wait()
                gather(kn, rn).start()

            if k == UNROLL - 1:
                pl.when(g < last_g)(prefetch_gather)
            else:
                prefetch_gather()

            gather(k % NEB, r).wait()

            cur = rows[r]
            eb = ebufs[k % NEB]

            def scale_body(grp, c2):
                w16 = lax.bitcast_convert_type(
                    eb[2, pl.ds(grp * 16, 16)], jnp.float32)
                for l in range(16):
                    w = w16[l]
                    e = grp * 16 + l
                    for j in range(D // 16):
                        sl = pl.ds(j * 16, 16)
                        cur[e, sl] = cur[e, sl] * w
                return c2

            lax.fori_loop(0, CH // 16, scale_body, 0)

            def drain_prev():
                scatter_desc((k - 1) % NEB, (k - 1) % NRB).wait()

            if k == 0:
                pl.when(g > 0)(drain_prev)
            else:
                drain_prev()

            def prefetch_record():
                edge_load(t + 3, (k + 3) % NEB).start()

            if k >= UNROLL - 3:
                pl.when(g < last_g)(prefetch_record)
            else:
                prefetch_record()

            pltpu.async_copy(cur, agg.at[eb.at[1]], ssems[r], add=True)
        return carry

    lax.fori_loop(0, CHW // UNROLL, body, 0)
    scatter_desc((CHW - 1) % NEB, (CHW - 1) % NRB).wait()
    plsc.subcore_barrier()

    pltpu.sync_copy(agg.at[pl.ds(r0, ROWS_PER_TILE)],
                    out_hbm.at[cid, pl.ds(r0, ROWS_PER_TILE)])


_edge_kernel = functools.partial(
    pl.kernel,
    out_type=jax.ShapeDtypeStruct((NC, NPAD, D), jnp.float32),
    mesh=plsc.VectorSubcoreMesh(core_axis_name="c", subcore_axis_name="s"),
    scratch_types=(
        [pltpu.VMEM((3, CH), jnp.int32)] * NEB
        + [pltpu.VMEM((CH, D), jnp.float32)] * NRB
        + [pltpu.VMEM_SHARED((NPAD, D), jnp.float32)]
        + [pltpu.SemaphoreType.DMA] * (NEB + 2 * NRB)
    ),
)(_edge_body)


def _comb_body(p_ref, o_ref):
    o_ref[...] = jnp.tanh(p_ref[0] + p_ref[1])


def _combine(partials):
    return pl.pallas_call(
        _comb_body,
        grid=(N // MM_BLK,),
        in_specs=[pl.BlockSpec((NC, MM_BLK, D), lambda i: (0, i, 0))],
        out_specs=pl.BlockSpec((MM_BLK, D), lambda i: (i, 0)),
        out_shape=jax.ShapeDtypeStruct((N, D), jnp.float32),
    )(partials)


def kernel(x, edge_index, edge_weight, W, b):
    dst = edge_index[0].astype(jnp.int32)
    src = edge_index[1].astype(jnp.int32)
    npad = EPAD - E
    pad_idx = jnp.arange(npad, dtype=jnp.int32) % N
    src2 = jnp.concatenate([src, pad_idx]).reshape(NROW, CH)
    dst2 = jnp.concatenate([dst, pad_idx]).reshape(NROW, CH)
    wb2 = jnp.concatenate(
        [edge_weight.view(jnp.int32),
         jnp.zeros((npad,), jnp.int32)]).reshape(NROW, CH)
    edata = jnp.stack([src2, dst2, wb2], axis=1)
    support = _support_matmul(x, W, b.reshape(1, D))
    zeros = jnp.zeros((ROWS_PER_TILE, D), jnp.float32)
    partials = _edge_kernel(support, edata, zeros)
    return _combine(partials)
